# Initial kernel scaffold; baseline (speedup 1.0000x reference)
import jax
import jax.numpy as jnp
from jax.experimental import pallas as pl


def kernel(x, edge_index, W1, W2, W3):
    raise NotImplementedError("write your pallas kernel here")


# speedup vs baseline: 16.2013x; 16.2013x over previous
import functools

import jax
import jax.numpy as jnp
from jax import lax
from jax.experimental import pallas as pl
from jax.experimental.pallas import tpu as pltpu
from jax.experimental.pallas import tpu_sc as plsc

N_NODES = 10000
N_EDGES = 320000
IN_FEAT = 128
HIDDEN = 128
NUM_CLASSES = 16

NC = 2
NS = 16
NW = NC * NS
EPT = N_EDGES // NW
ROWS_PT = 624
ROWS_REM = N_NODES - NS * ROWS_PT
REM_BASE = NS * ROWS_PT

_MESH = plsc.VectorSubcoreMesh(core_axis_name="c", subcore_axis_name="s")


@functools.partial(
    pl.kernel,
    out_type=jax.ShapeDtypeStruct((NW, N_NODES), jnp.float32),
    mesh=_MESH,
    scratch_types=[
        pltpu.VMEM((EPT,), jnp.int32),
        pltpu.VMEM((N_NODES,), jnp.float32),
    ],
    compiler_params=pltpu.CompilerParams(needs_layout_passes=False),
)
def _deg_kernel(dst_hbm, out_hbm, idx_v, deg_v):
    c = lax.axis_index("c")
    s = lax.axis_index("s")
    wid = c * NS + s
    base = wid * EPT
    pltpu.sync_copy(dst_hbm.at[pl.ds(base, EPT)], idx_v)

    zeros16 = jnp.zeros((16,), jnp.float32)

    def zero_body(i, carry):
        deg_v[pl.ds(i * 16, 16)] = zeros16
        return carry

    lax.fori_loop(0, N_NODES // 16, zero_body, 0)

    ones16 = jnp.ones((16,), jnp.float32)

    def add_body(i, carry):
        idx = idx_v[pl.ds(i * 16, 16)]
        plsc.addupdate_scatter(deg_v, [idx], ones16)
        return carry

    lax.fori_loop(0, EPT // 16, add_body, 0)
    pltpu.sync_copy(deg_v, out_hbm.at[wid])


def _make_edge_scatter(feat, k_chunk, tc_tiling):
    nchunks = EPT // k_chunk
    assert nchunks * k_chunk == EPT and k_chunk % 8 == 0 and k_chunk <= 128

    @functools.partial(
        pl.kernel,
        out_type=jax.ShapeDtypeStruct((NC, N_NODES, feat), jnp.float32),
        mesh=_MESH,
        compiler_params=pltpu.CompilerParams(use_tc_tiling_on_sc=tc_tiling),
        scratch_types=[
            pltpu.VMEM((k_chunk,), jnp.int32),
            pltpu.VMEM((k_chunk,), jnp.int32),
            pltpu.VMEM((k_chunk, feat), jnp.float32),
            pltpu.VMEM_SHARED((N_NODES, feat), jnp.float32),
            pltpu.SemaphoreType.DMA,
        ],
    )
    def edge_kernel(y_hbm, src_hbm, dst_hbm, zeros_hbm, out_hbm,
                    sidx, didx, rows, acc, gsem):
        c = lax.axis_index("c")
        s = lax.axis_index("s")
        wid = c * NS + s
        ebase = wid * EPT
        rbase = s * ROWS_PT

        pltpu.sync_copy(zeros_hbm, acc.at[pl.ds(rbase, ROWS_PT)])

        @pl.when(s == NS - 1)
        def _():
            pltpu.sync_copy(zeros_hbm.at[pl.ds(0, ROWS_REM)],
                            acc.at[pl.ds(REM_BASE, ROWS_REM)])

        plsc.subcore_barrier()

        def body(g, carry):
            off = ebase + g * k_chunk
            pltpu.sync_copy(src_hbm.at[pl.ds(off, k_chunk)], sidx)
            pltpu.sync_copy(dst_hbm.at[pl.ds(off, k_chunk)], didx)
            pltpu.async_copy(y_hbm.at[sidx], rows, gsem).wait()
            pltpu.sync_copy(rows, acc.at[didx], add=True)
            return carry

        lax.fori_loop(0, nchunks, body, 0)
        plsc.subcore_barrier()
        pltpu.sync_copy(acc.at[pl.ds(rbase, ROWS_PT)],
                        out_hbm.at[c, pl.ds(rbase, ROWS_PT)])

        @pl.when(s == NS - 1)
        def _():
            pltpu.sync_copy(acc.at[pl.ds(REM_BASE, ROWS_REM)],
                            out_hbm.at[c, pl.ds(REM_BASE, ROWS_REM)])

    return edge_kernel


_edge_scatter_128 = _make_edge_scatter(HIDDEN, 80, True)
_edge_scatter_16 = _make_edge_scatter(NUM_CLASSES, 80, False)


_BLK = 1000
_GRID = N_NODES // _BLK

_DINV_SPEC = pl.BlockSpec((1, 1, _BLK), lambda i: (i, 0, 0))


def _tc_dinv_body(degp_ref, o_ref):
    deg = jnp.sum(degp_ref[...], axis=0) + 1.0
    o_ref[...] = lax.rsqrt(deg)[None, :]


def _tc_dinv(degp):
    return pl.pallas_call(
        _tc_dinv_body,
        grid=(1,),
        in_specs=[pl.BlockSpec((NW, N_NODES), lambda i: (0, 0))],
        out_specs=pl.BlockSpec((1, N_NODES), lambda i: (0, 0)),
        out_shape=jax.ShapeDtypeStruct((1, N_NODES), jnp.float32),
    )(degp)


def _dinv_block(dinv_ref):
    return dinv_ref[...].reshape(_BLK)


def _tc_y1_body(x_ref, w1_ref, dinv_ref, y_ref):
    dinv = _dinv_block(dinv_ref)
    y_ref[...] = jnp.dot(x_ref[...], w1_ref[...],
                         preferred_element_type=jnp.float32) * dinv[:, None]


def _tc_y1(x, w1, dinv):
    return pl.pallas_call(
        _tc_y1_body,
        grid=(_GRID,),
        in_specs=[
            pl.BlockSpec((_BLK, IN_FEAT), lambda i: (i, 0)),
            pl.BlockSpec((IN_FEAT, HIDDEN), lambda i: (0, 0)),
            _DINV_SPEC,
        ],
        out_specs=pl.BlockSpec((_BLK, HIDDEN), lambda i: (i, 0)),
        out_shape=jax.ShapeDtypeStruct((N_NODES, HIDDEN), jnp.float32),
    )(x, w1, dinv)


def _tc_mid_body(acc_ref, y1_ref, dinv_ref, w2_ref, w3_ref, y2_ref):
    dinv = _dinv_block(dinv_ref)
    h = (acc_ref[0] + acc_ref[1] + y1_ref[...]) * dinv[:, None]
    h = jnp.maximum(h, 0.0)
    w23 = jnp.dot(w2_ref[...], w3_ref[...], preferred_element_type=jnp.float32)
    y2_ref[...] = jnp.dot(h, w23, preferred_element_type=jnp.float32) * dinv[:, None]


def _tc_mid(acc, y1, dinv, w2, w3):
    return pl.pallas_call(
        _tc_mid_body,
        grid=(_GRID,),
        in_specs=[
            pl.BlockSpec((NC, _BLK, HIDDEN), lambda i: (0, i, 0)),
            pl.BlockSpec((_BLK, HIDDEN), lambda i: (i, 0)),
            _DINV_SPEC,
            pl.BlockSpec((HIDDEN, HIDDEN), lambda i: (0, 0)),
            pl.BlockSpec((HIDDEN, NUM_CLASSES), lambda i: (0, 0)),
        ],
        out_specs=pl.BlockSpec((_BLK, NUM_CLASSES), lambda i: (i, 0)),
        out_shape=jax.ShapeDtypeStruct((N_NODES, NUM_CLASSES), jnp.float32),
    )(acc, y1, dinv, w2, w3)


def _tc_out_body(acc_ref, y2_ref, dinv_ref, o_ref):
    dinv = _dinv_block(dinv_ref)
    o_ref[...] = (acc_ref[0] + acc_ref[1] + y2_ref[...]) * dinv[:, None]


def _tc_out(acc, y2, dinv):
    return pl.pallas_call(
        _tc_out_body,
        grid=(_GRID,),
        in_specs=[
            pl.BlockSpec((NC, _BLK, NUM_CLASSES), lambda i: (0, i, 0)),
            pl.BlockSpec((_BLK, NUM_CLASSES), lambda i: (i, 0)),
            _DINV_SPEC,
        ],
        out_specs=pl.BlockSpec((_BLK, NUM_CLASSES), lambda i: (i, 0)),
        out_shape=jax.ShapeDtypeStruct((N_NODES, NUM_CLASSES), jnp.float32),
    )(acc, y2, dinv)


def kernel(x, edge_index, W1, W2, W3):
    src = edge_index[0].astype(jnp.int32)
    dst = edge_index[1].astype(jnp.int32)
    zeros128 = jnp.zeros((ROWS_PT, HIDDEN), jnp.float32)
    zeros16 = jnp.zeros((ROWS_PT, NUM_CLASSES), jnp.float32)

    degp = _deg_kernel(dst)
    dinv = _tc_dinv(degp)
    dinv = dinv.reshape(_GRID, 1, _BLK)
    y1 = _tc_y1(x, W1, dinv)
    acc1 = _edge_scatter_128(y1, src, dst, zeros128)
    y2 = _tc_mid(acc1, y1, dinv, W2, W3)
    acc2 = _edge_scatter_16(y2, src, dst, zeros16)
    return _tc_out(acc2, y2, dinv)
